# Initial kernel scaffold; baseline (speedup 1.0000x reference)
import jax
import jax.numpy as jnp
from jax.experimental import pallas as pl


def kernel(lv, edge_index1, edge_index2, W_self1, W_nbr1, b1, W_self2, W_nbr2, b2):
    raise NotImplementedError("write your pallas kernel here")


# speedup vs baseline: 5.0144x; 5.0144x over previous
import functools

import jax
import jax.numpy as jnp
from jax import lax
from jax.experimental import pallas as pl
from jax.experimental.pallas import tpu as pltpu
from jax.experimental.pallas import tpu_sc as plsc

N = 10000
E = 320000
F = 128

NUM_SC = 2
NUM_TILES = 16
NW = NUM_SC * NUM_TILES
EDGES_PER_W = E // NW
CHUNK = 80
NCHUNKS = EDGES_PER_W // CHUNK
NPAD = 10240
ROWS_PER_TILE = NPAD // NUM_TILES


def _sc_segment_sum(x, src, dst, zeros_nf):
    mesh = plsc.VectorSubcoreMesh(core_axis_name="c", subcore_axis_name="s")

    @functools.partial(
        pl.kernel,
        out_type=jax.ShapeDtypeStruct((NUM_SC * NPAD, F), jnp.float32),
        mesh=mesh,
        scratch_types=[
            pltpu.VMEM((CHUNK,), jnp.int32),
            pltpu.VMEM((CHUNK,), jnp.int32),
            pltpu.VMEM((CHUNK, F), jnp.float32),
            pltpu.VMEM_SHARED((NPAD, F), jnp.float32),
            pltpu.SemaphoreType.DMA,
        ],
    )
    def seg_sum(x_hbm, src_hbm, dst_hbm, z_hbm, out_hbm, idx_s, idx_d, rows, acc, sem):
        c = lax.axis_index("c")
        s = lax.axis_index("s")
        wid = c * NUM_TILES + s

        row0 = s * ROWS_PER_TILE
        pltpu.sync_copy(z_hbm.at[pl.ds(row0, ROWS_PER_TILE)],
                        acc.at[pl.ds(row0, ROWS_PER_TILE)])
        plsc.subcore_barrier()

        ebase = wid * EDGES_PER_W

        def body(i, carry):
            off = ebase + i * CHUNK
            pltpu.sync_copy(src_hbm.at[pl.ds(off, CHUNK)], idx_s)
            pltpu.sync_copy(dst_hbm.at[pl.ds(off, CHUNK)], idx_d)
            pltpu.async_copy(x_hbm.at[idx_s], rows, sem).wait()
            pltpu.sync_copy(rows, acc.at[idx_d], add=True)
            return carry

        lax.fori_loop(0, NCHUNKS, body, 0, unroll=False)

        plsc.subcore_barrier()
        pltpu.sync_copy(acc.at[pl.ds(row0, ROWS_PER_TILE)],
                        out_hbm.at[pl.ds(c * NPAD + row0, ROWS_PER_TILE)])

    return seg_sum(x, src, dst, zeros_nf)


def _tc_layer(x, p0, p1, w_self, w_nbr, b, residual=None):
    blk = 1000
    grid = (N // blk,)

    def body(*refs):
        if residual is None:
            x_r, p0_r, p1_r, ws_r, wn_r, b_r, o_r = refs
        else:
            x_r, p0_r, p1_r, ws_r, wn_r, b_r, res_r, o_r = refs
        agg = p0_r[...] + p1_r[...]
        acc = jnp.dot(x_r[...], ws_r[...], preferred_element_type=jnp.float32)
        acc = acc + jnp.dot(agg, wn_r[...], preferred_element_type=jnp.float32)
        acc = jnp.maximum(acc + b_r[...], 0.0)
        if residual is not None:
            acc = acc + res_r[...]
        o_r[...] = acc

    row_spec = pl.BlockSpec((blk, F), lambda i: (i, 0))
    w_spec = pl.BlockSpec((F, F), lambda i: (0, 0))
    b_spec = pl.BlockSpec((1, F), lambda i: (0, 0))
    in_specs = [row_spec, row_spec, row_spec, w_spec, w_spec, b_spec]
    args = [x, p0, p1, w_self, w_nbr, b.reshape(1, F)]
    if residual is not None:
        in_specs.append(row_spec)
        args.append(residual)

    return pl.pallas_call(
        body,
        grid=grid,
        in_specs=in_specs,
        out_specs=row_spec,
        out_shape=jax.ShapeDtypeStruct((N, F), jnp.float32),
    )(*args)


def kernel(lv, edge_index1, edge_index2, W_self1, W_nbr1, b1, W_self2, W_nbr2, b2):
    src1, dst1 = edge_index1[0], edge_index1[1]
    src2, dst2 = edge_index2[0], edge_index2[1]
    zeros_nf = jnp.zeros((NPAD, F), jnp.float32)

    p = _sc_segment_sum(lv, src1, dst1, zeros_nf)
    h = _tc_layer(lv, p[:N], p[NPAD:NPAD + N], W_self1, W_nbr1, b1)
    q = _sc_segment_sum(h, src2, dst2, zeros_nf)
    out = _tc_layer(h, q[:N], q[NPAD:NPAD + N], W_self2, W_nbr2, b2, residual=lv)
    return out


# speedup vs baseline: 8.9903x; 1.7929x over previous
import functools

import jax
import jax.numpy as jnp
from jax import lax
from jax.experimental import pallas as pl
from jax.experimental.pallas import tpu as pltpu
from jax.experimental.pallas import tpu_sc as plsc

N = 10000
E = 320000
F = 128

NUM_SC = 2
NUM_TILES = 16
NW = NUM_SC * NUM_TILES
EDGES_PER_W = E // NW
CHUNK = 80
NCHUNKS = EDGES_PER_W // CHUNK
NPAD = 10240
ROWS_PER_TILE = NPAD // NUM_TILES


def _sc_segment_sum(x, src, dst, zeros_nf):
    mesh = plsc.VectorSubcoreMesh(core_axis_name="c", subcore_axis_name="s")

    @functools.partial(
        pl.kernel,
        out_type=jax.ShapeDtypeStruct((NUM_SC * NPAD, F), jnp.float32),
        mesh=mesh,
        scratch_types=[
            pltpu.VMEM((CHUNK,), jnp.int32),
            pltpu.VMEM((CHUNK,), jnp.int32),
            pltpu.VMEM((CHUNK,), jnp.int32),
            pltpu.VMEM((CHUNK,), jnp.int32),
            pltpu.VMEM((CHUNK, F), jnp.float32),
            pltpu.VMEM((CHUNK, F), jnp.float32),
            pltpu.VMEM_SHARED((NPAD, F), jnp.float32),
            pltpu.SemaphoreType.DMA,
            pltpu.SemaphoreType.DMA,
            pltpu.SemaphoreType.DMA,
            pltpu.SemaphoreType.DMA,
            pltpu.SemaphoreType.DMA,
            pltpu.SemaphoreType.DMA,
            pltpu.SemaphoreType.DMA,
            pltpu.SemaphoreType.DMA,
        ],
    )
    def seg_sum(x_hbm, src_hbm, dst_hbm, z_hbm, out_hbm,
                idx_s0, idx_s1, idx_d0, idx_d1, rows0, rows1, acc,
                isem0, isem1, dsem0, dsem1, gsem0, gsem1, ssem0, ssem1):
        c = lax.axis_index("c")
        s = lax.axis_index("s")
        wid = c * NUM_TILES + s
        idx_s = (idx_s0, idx_s1)
        idx_d = (idx_d0, idx_d1)
        rows = (rows0, rows1)
        isem = (isem0, isem1)
        dsem = (dsem0, dsem1)
        gsem = (gsem0, gsem1)
        ssem = (ssem0, ssem1)

        row0 = s * ROWS_PER_TILE
        pltpu.sync_copy(z_hbm.at[pl.ds(row0, ROWS_PER_TILE)],
                        acc.at[pl.ds(row0, ROWS_PER_TILE)])
        plsc.subcore_barrier()

        ebase = wid * EDGES_PER_W

        def start_src(i, b):
            off = ebase + i * CHUNK
            pltpu.async_copy(src_hbm.at[pl.ds(off, CHUNK)], idx_s[b], isem[b])

        def start_dst(i, b):
            off = ebase + i * CHUNK
            pltpu.async_copy(dst_hbm.at[pl.ds(off, CHUNK)], idx_d[b], dsem[b])

        def wait_src(b):
            pltpu.make_async_copy(src_hbm.at[pl.ds(0, CHUNK)], idx_s[b], isem[b]).wait()

        def wait_dst(b):
            pltpu.make_async_copy(dst_hbm.at[pl.ds(0, CHUNK)], idx_d[b], dsem[b]).wait()

        def wait_scatter(b):
            pltpu.make_async_copy(rows[b], acc.at[idx_d[b]], ssem[b]).wait()

        def run_chunk(i, b, first, prefetch):
            if not first:
                wait_scatter(b)
            start_dst(i, b)
            wait_src(b)
            g = pltpu.async_copy(x_hbm.at[idx_s[b]], rows[b], gsem[b])
            g.wait()
            if prefetch:
                start_src(i + 2, b)
            wait_dst(b)
            pltpu.async_copy(rows[b], acc.at[idx_d[b]], ssem[b], add=True)

        start_src(0, 0)
        start_src(1, 1)
        run_chunk(0, 0, True, True)
        run_chunk(1, 1, True, True)

        def body(k, carry):
            for b in range(2):
                i = 2 * k + b
                wait_scatter(b)
                start_dst(i, b)
                wait_src(b)
                pltpu.async_copy(x_hbm.at[idx_s[b]], rows[b], gsem[b]).wait()
                if b == 0:
                    start_src(i + 2, b)
                else:
                    @pl.when(k < (NCHUNKS - 1) // 2 - 1)
                    def _():
                        start_src(i + 2, b)
                wait_dst(b)
                pltpu.async_copy(rows[b], acc.at[idx_d[b]], ssem[b], add=True)
            return carry

        lax.fori_loop(1, (NCHUNKS - 1) // 2, body, 0, unroll=False)

        run_chunk(NCHUNKS - 1, 0, False, False)
        wait_scatter(1)
        wait_scatter(0)

        plsc.subcore_barrier()
        pltpu.sync_copy(acc.at[pl.ds(row0, ROWS_PER_TILE)],
                        out_hbm.at[pl.ds(c * NPAD + row0, ROWS_PER_TILE)])

    return seg_sum(x, src, dst, zeros_nf)


def _tc_layer(x, p0, p1, w_self, w_nbr, b, residual=None):
    blk = 1000
    grid = (N // blk,)

    def body(*refs):
        if residual is None:
            x_r, p0_r, p1_r, ws_r, wn_r, b_r, o_r = refs
        else:
            x_r, p0_r, p1_r, ws_r, wn_r, b_r, res_r, o_r = refs
        agg = p0_r[...] + p1_r[...]
        acc = jnp.dot(x_r[...], ws_r[...], preferred_element_type=jnp.float32)
        acc = acc + jnp.dot(agg, wn_r[...], preferred_element_type=jnp.float32)
        acc = jnp.maximum(acc + b_r[...], 0.0)
        if residual is not None:
            acc = acc + res_r[...]
        o_r[...] = acc

    row_spec = pl.BlockSpec((blk, F), lambda i: (i, 0))
    w_spec = pl.BlockSpec((F, F), lambda i: (0, 0))
    b_spec = pl.BlockSpec((1, F), lambda i: (0, 0))
    in_specs = [row_spec, row_spec, row_spec, w_spec, w_spec, b_spec]
    args = [x, p0, p1, w_self, w_nbr, b.reshape(1, F)]
    if residual is not None:
        in_specs.append(row_spec)
        args.append(residual)

    return pl.pallas_call(
        body,
        grid=grid,
        in_specs=in_specs,
        out_specs=row_spec,
        out_shape=jax.ShapeDtypeStruct((N, F), jnp.float32),
    )(*args)


def kernel(lv, edge_index1, edge_index2, W_self1, W_nbr1, b1, W_self2, W_nbr2, b2):
    src1, dst1 = edge_index1[0], edge_index1[1]
    src2, dst2 = edge_index2[0], edge_index2[1]
    zeros_nf = jnp.zeros((NPAD, F), jnp.float32)

    p = _sc_segment_sum(lv, src1, dst1, zeros_nf)
    h = _tc_layer(lv, p[:N], p[NPAD:NPAD + N], W_self1, W_nbr1, b1)
    q = _sc_segment_sum(h, src2, dst2, zeros_nf)
    out = _tc_layer(h, q[:N], q[NPAD:NPAD + N], W_self2, W_nbr2, b2, residual=lv)
    return out


# speedup vs baseline: 12.7124x; 1.4140x over previous
import functools

import jax
import jax.numpy as jnp
from jax import lax
from jax.experimental import pallas as pl
from jax.experimental.pallas import tpu as pltpu
from jax.experimental.pallas import tpu_sc as plsc

N = 10000
E = 320000
F = 128

NUM_SC = 2
NUM_TILES = 16
NW = NUM_SC * NUM_TILES
EDGES_PER_W = E // NW
CHUNK = 80
NCHUNKS = EDGES_PER_W // CHUNK
NPAD = 10240
ROWS_PER_TILE = NPAD // NUM_TILES


def _sc_segment_sum(x, src, dst, zeros_nf):
    mesh = plsc.VectorSubcoreMesh(core_axis_name="c", subcore_axis_name="s")

    @functools.partial(
        pl.kernel,
        out_type=jax.ShapeDtypeStruct((NUM_SC * NPAD, F), jnp.float32),
        mesh=mesh,
        scratch_types=(
            [pltpu.VMEM((CHUNK,), jnp.int32)] * 6
            + [pltpu.VMEM((CHUNK, F), jnp.float32)] * 3
            + [pltpu.VMEM_SHARED((NPAD, F), jnp.float32)]
            + [pltpu.SemaphoreType.DMA] * 12
        ),
    )
    def seg_sum(x_hbm, src_hbm, dst_hbm, z_hbm, out_hbm,
                idx_s0, idx_s1, idx_s2, idx_d0, idx_d1, idx_d2,
                rows0, rows1, rows2, acc,
                isem0, isem1, isem2, dsem0, dsem1, dsem2,
                gsem0, gsem1, gsem2, ssem0, ssem1, ssem2):
        c = lax.axis_index("c")
        s = lax.axis_index("s")
        wid = c * NUM_TILES + s
        idx_s = (idx_s0, idx_s1, idx_s2)
        idx_d = (idx_d0, idx_d1, idx_d2)
        rows = (rows0, rows1, rows2)
        isem = (isem0, isem1, isem2)
        dsem = (dsem0, dsem1, dsem2)
        gsem = (gsem0, gsem1, gsem2)
        ssem = (ssem0, ssem1, ssem2)

        row0 = s * ROWS_PER_TILE
        pltpu.sync_copy(z_hbm.at[pl.ds(row0, ROWS_PER_TILE)],
                        acc.at[pl.ds(row0, ROWS_PER_TILE)])
        plsc.subcore_barrier()

        ebase = wid * EDGES_PER_W

        def start_src(i, b):
            pltpu.async_copy(src_hbm.at[pl.ds(ebase + i * CHUNK, CHUNK)],
                             idx_s[b], isem[b])

        def start_dst(i, b):
            pltpu.async_copy(dst_hbm.at[pl.ds(ebase + i * CHUNK, CHUNK)],
                             idx_d[b], dsem[b])

        def wait_src(b):
            pltpu.make_async_copy(src_hbm.at[pl.ds(0, CHUNK)], idx_s[b], isem[b]).wait()

        def wait_dst(b):
            pltpu.make_async_copy(dst_hbm.at[pl.ds(0, CHUNK)], idx_d[b], dsem[b]).wait()

        def start_gather(b):
            pltpu.async_copy(x_hbm.at[idx_s[b]], rows[b], gsem[b])

        def wait_gather(b):
            pltpu.make_async_copy(x_hbm.at[idx_s[b]], rows[b], gsem[b]).wait()

        def start_scatter(b):
            pltpu.async_copy(rows[b], acc.at[idx_d[b]], ssem[b], add=True)

        def wait_scatter(b):
            pltpu.make_async_copy(rows[b], acc.at[idx_d[b]], ssem[b]).wait()

        def steady(i, b, k_pred=None, skip_c_wait=False):
            bp = (b + 2) % 3
            if not skip_c_wait:
                wait_scatter(b)
            start_dst(i, b)
            wait_src(b)
            start_gather(b)
            wait_gather(bp)
            if k_pred is None:
                start_src(i + 2, bp)
            elif k_pred is not False:
                @pl.when(k_pred)
                def _():
                    start_src(i + 2, bp)
            wait_dst(bp)
            start_scatter(bp)

        start_src(0, 0)
        start_dst(0, 0)
        start_src(1, 1)
        start_dst(1, 1)
        start_src(2, 2)
        wait_src(0)
        start_gather(0)
        wait_src(1)
        start_gather(1)
        wait_gather(0)
        start_src(3, 0)
        wait_dst(0)
        start_scatter(0)

        steady(2, 2, skip_c_wait=True)
        steady(3, 0)
        steady(4, 1)

        def body(k, carry):
            i0 = 5 + 3 * k
            steady(i0, 2)
            steady(i0 + 1, 0, k_pred=(k < (NCHUNKS - 5) // 3 - 1))
            steady(i0 + 2, 1, k_pred=(k < (NCHUNKS - 5) // 3 - 1))
            return carry

        lax.fori_loop(0, (NCHUNKS - 5) // 3, body, 0, unroll=False)

        wait_gather(1)
        wait_dst(1)
        start_scatter(1)
        wait_scatter(2)
        wait_scatter(0)
        wait_scatter(1)

        plsc.subcore_barrier()
        pltpu.sync_copy(acc.at[pl.ds(row0, ROWS_PER_TILE)],
                        out_hbm.at[pl.ds(c * NPAD + row0, ROWS_PER_TILE)])

    return seg_sum(x, src, dst, zeros_nf)


def _tc_layer(x, p0, p1, w_self, w_nbr, b, residual=None):
    blk = 1000
    grid = (N // blk,)

    def body(*refs):
        if residual is None:
            x_r, p0_r, p1_r, ws_r, wn_r, b_r, o_r = refs
        else:
            x_r, p0_r, p1_r, ws_r, wn_r, b_r, res_r, o_r = refs
        agg = p0_r[...] + p1_r[...]
        acc = jnp.dot(x_r[...], ws_r[...], preferred_element_type=jnp.float32)
        acc = acc + jnp.dot(agg, wn_r[...], preferred_element_type=jnp.float32)
        acc = jnp.maximum(acc + b_r[...], 0.0)
        if residual is not None:
            acc = acc + res_r[...]
        o_r[...] = acc

    row_spec = pl.BlockSpec((blk, F), lambda i: (i, 0))
    w_spec = pl.BlockSpec((F, F), lambda i: (0, 0))
    b_spec = pl.BlockSpec((1, F), lambda i: (0, 0))
    in_specs = [row_spec, row_spec, row_spec, w_spec, w_spec, b_spec]
    args = [x, p0, p1, w_self, w_nbr, b.reshape(1, F)]
    if residual is not None:
        in_specs.append(row_spec)
        args.append(residual)

    return pl.pallas_call(
        body,
        grid=grid,
        in_specs=in_specs,
        out_specs=row_spec,
        out_shape=jax.ShapeDtypeStruct((N, F), jnp.float32),
    )(*args)


def kernel(lv, edge_index1, edge_index2, W_self1, W_nbr1, b1, W_self2, W_nbr2, b2):
    src1, dst1 = edge_index1[0], edge_index1[1]
    src2, dst2 = edge_index2[0], edge_index2[1]
    zeros_nf = jnp.zeros((NPAD, F), jnp.float32)

    p = _sc_segment_sum(lv, src1, dst1, zeros_nf)
    h = _tc_layer(lv, p[:N], p[NPAD:NPAD + N], W_self1, W_nbr1, b1)
    q = _sc_segment_sum(h, src2, dst2, zeros_nf)
    out = _tc_layer(h, q[:N], q[NPAD:NPAD + N], W_self2, W_nbr2, b2, residual=lv)
    return out


# speedup vs baseline: 14.1725x; 1.1149x over previous
import functools

import jax
import jax.numpy as jnp
from jax import lax
from jax.experimental import pallas as pl
from jax.experimental.pallas import tpu as pltpu
from jax.experimental.pallas import tpu_sc as plsc

N = 10000
E = 320000
F = 128

NUM_SC = 2
NUM_TILES = 16
NW = NUM_SC * NUM_TILES
EDGES_PER_W = E // NW
CHUNK = 80
NCHUNKS = EDGES_PER_W // CHUNK
NPAD = 10240
ROWS_PER_TILE = NPAD // NUM_TILES


def _sc_segment_sum(x, edges_flat, zeros_blk):
    mesh = plsc.VectorSubcoreMesh(core_axis_name="c", subcore_axis_name="s")

    @functools.partial(
        pl.kernel,
        out_type=[jax.ShapeDtypeStruct((NPAD, F), jnp.float32),
                  jax.ShapeDtypeStruct((NPAD, F), jnp.float32)],
        mesh=mesh,
        scratch_types=(
            [pltpu.VMEM((CHUNK,), jnp.int32)] * 6
            + [pltpu.VMEM((CHUNK, F), jnp.float32)] * 3
            + [pltpu.VMEM_SHARED((NPAD, F), jnp.float32)]
            + [pltpu.SemaphoreType.DMA] * 12
        ),
    )
    def seg_sum(x_hbm, e_hbm, z_hbm, out0_hbm, out1_hbm,
                idx_s0, idx_s1, idx_s2, idx_d0, idx_d1, idx_d2,
                rows0, rows1, rows2, acc,
                isem0, isem1, isem2, dsem0, dsem1, dsem2,
                gsem0, gsem1, gsem2, ssem0, ssem1, ssem2):
        c = lax.axis_index("c")
        s = lax.axis_index("s")
        wid = c * NUM_TILES + s
        idx_s = (idx_s0, idx_s1, idx_s2)
        idx_d = (idx_d0, idx_d1, idx_d2)
        rows = (rows0, rows1, rows2)
        isem = (isem0, isem1, isem2)
        dsem = (dsem0, dsem1, dsem2)
        gsem = (gsem0, gsem1, gsem2)
        ssem = (ssem0, ssem1, ssem2)

        row0 = s * ROWS_PER_TILE
        pltpu.sync_copy(z_hbm, rows0)
        for t in range(ROWS_PER_TILE // CHUNK):
            pltpu.sync_copy(rows0, acc.at[pl.ds(row0 + t * CHUNK, CHUNK)])
        plsc.subcore_barrier()

        ebase = wid * EDGES_PER_W

        def start_src(i, b):
            pltpu.async_copy(e_hbm.at[pl.ds(ebase + i * CHUNK, CHUNK)],
                             idx_s[b], isem[b])

        def start_dst(i, b):
            pltpu.async_copy(e_hbm.at[pl.ds(E + ebase + i * CHUNK, CHUNK)],
                             idx_d[b], dsem[b])

        def wait_src(b):
            pltpu.make_async_copy(e_hbm.at[pl.ds(0, CHUNK)], idx_s[b], isem[b]).wait()

        def wait_dst(b):
            pltpu.make_async_copy(e_hbm.at[pl.ds(0, CHUNK)], idx_d[b], dsem[b]).wait()

        def start_gather(b):
            pltpu.async_copy(x_hbm.at[idx_s[b]], rows[b], gsem[b])

        def wait_gather(b):
            pltpu.make_async_copy(x_hbm.at[idx_s[b]], rows[b], gsem[b]).wait()

        def start_scatter(b):
            pltpu.async_copy(rows[b], acc.at[idx_d[b]], ssem[b], add=True)

        def wait_scatter(b):
            pltpu.make_async_copy(rows[b], acc.at[idx_d[b]], ssem[b]).wait()

        def steady(i, b, k_pred=None, skip_c_wait=False):
            bp = (b + 2) % 3
            if not skip_c_wait:
                wait_scatter(b)
            start_dst(i, b)
            wait_src(b)
            start_gather(b)
            wait_gather(bp)
            if k_pred is None:
                start_src(i + 2, bp)
            else:
                @pl.when(k_pred)
                def _():
                    start_src(i + 2, bp)
            wait_dst(bp)
            start_scatter(bp)

        start_src(0, 0)
        start_dst(0, 0)
        start_src(1, 1)
        start_dst(1, 1)
        start_src(2, 2)
        wait_src(0)
        start_gather(0)
        wait_src(1)
        start_gather(1)
        wait_gather(0)
        start_src(3, 0)
        wait_dst(0)
        start_scatter(0)

        steady(2, 2, skip_c_wait=True)
        steady(3, 0)
        steady(4, 1)

        def body(k, carry):
            i0 = 5 + 3 * k
            steady(i0, 2)
            steady(i0 + 1, 0, k_pred=(k < (NCHUNKS - 5) // 3 - 1))
            steady(i0 + 2, 1, k_pred=(k < (NCHUNKS - 5) // 3 - 1))
            return carry

        lax.fori_loop(0, (NCHUNKS - 5) // 3, body, 0, unroll=False)

        wait_gather(1)
        wait_dst(1)
        start_scatter(1)
        wait_scatter(2)
        wait_scatter(0)
        wait_scatter(1)

        plsc.subcore_barrier()
        stripe = pl.ds(row0, ROWS_PER_TILE)

        @pl.when(c == 0)
        def _():
            pltpu.sync_copy(acc.at[stripe], out0_hbm.at[stripe])

        @pl.when(c == 1)
        def _():
            pltpu.sync_copy(acc.at[stripe], out1_hbm.at[stripe])

    return seg_sum(x, edges_flat, zeros_blk)


def _tc_layer(x, p0, p1, w_self, w_nbr, b, residual=None):
    blk = 1000
    grid = (N // blk,)

    def body(*refs):
        if residual is None:
            x_r, p0_r, p1_r, ws_r, wn_r, b_r, o_r = refs
        else:
            x_r, p0_r, p1_r, ws_r, wn_r, b_r, res_r, o_r = refs
        agg = p0_r[...] + p1_r[...]
        acc = jnp.dot(x_r[...], ws_r[...], preferred_element_type=jnp.float32)
        acc = acc + jnp.dot(agg, wn_r[...], preferred_element_type=jnp.float32)
        acc = jnp.maximum(acc + b_r[...], 0.0)
        if residual is not None:
            acc = acc + res_r[...]
        o_r[...] = acc

    row_spec = pl.BlockSpec((blk, F), lambda i: (i, 0))
    w_spec = pl.BlockSpec((F, F), lambda i: (0, 0))
    b_spec = pl.BlockSpec((1, F), lambda i: (0, 0))
    in_specs = [row_spec, row_spec, row_spec, w_spec, w_spec, b_spec]
    args = [x, p0, p1, w_self, w_nbr, b.reshape(1, F)]
    if residual is not None:
        in_specs.append(row_spec)
        args.append(residual)

    return pl.pallas_call(
        body,
        grid=grid,
        in_specs=in_specs,
        out_specs=row_spec,
        out_shape=jax.ShapeDtypeStruct((N, F), jnp.float32),
    )(*args)


def kernel(lv, edge_index1, edge_index2, W_self1, W_nbr1, b1, W_self2, W_nbr2, b2):
    e1 = edge_index1.reshape(2 * E)
    e2 = edge_index2.reshape(2 * E)
    zeros_blk = jnp.zeros((CHUNK, F), jnp.float32)

    p0, p1 = _sc_segment_sum(lv, e1, zeros_blk)
    h = _tc_layer(lv, p0, p1, W_self1, W_nbr1, b1)
    q0, q1 = _sc_segment_sum(h, e2, zeros_blk)
    out = _tc_layer(h, q0, q1, W_self2, W_nbr2, b2, residual=lv)
    return out
